# baseline (speedup 1.0000x reference)
import jax
import jax.numpy as jnp
from jax import lax
from jax.experimental import pallas as pl
from jax.experimental.pallas import tpu as pltpu
from jax.experimental.pallas import tpu_sc as plsc

VOCAB = 49408
HIDDEN = 768
SEQ = 77
BATCH = 1024

NUM_CORES = 2
NUM_SUBCORES = 16
NUM_WORKERS = NUM_CORES * NUM_SUBCORES
BATCHES_PER_WORKER = BATCH // NUM_WORKERS

FULL_ROWS = 72
TAIL = 8
BATCH_BLOCK = 8


def _embed_body(ids_hbm, tids_hbm, tok_hbm, out_hbm, tail_hbm,
                idx_v, tidx_v, rows_v, tail_v, sem, tsem):
    cid = lax.axis_index("c")
    sid = lax.axis_index("s")
    wid = sid * NUM_CORES + cid
    base_b = wid * BATCHES_PER_WORKER

    def batch_body(i, carry):
        gb = base_b + i
        pltpu.sync_copy(ids_hbm.at[gb], idx_v)
        pltpu.sync_copy(tids_hbm.at[gb], tidx_v)
        g = pltpu.async_copy(tok_hbm.at[idx_v], rows_v, sem)
        gt = pltpu.async_copy(tok_hbm.at[tidx_v], tail_v, tsem)
        g.wait()
        s = pltpu.async_copy(rows_v, out_hbm.at[gb], sem)
        gt.wait()
        st = pltpu.async_copy(tail_v, tail_hbm.at[gb], tsem)
        s.wait()
        st.wait()
        return carry

    lax.fori_loop(0, BATCHES_PER_WORKER, batch_body, 0)


def _add_body(x_ref, tail_ref, pos_ref, o_ref):
    pos = pos_ref[...]
    o_ref[:, :FULL_ROWS, :] = x_ref[:, :FULL_ROWS, :] + pos[None, :FULL_ROWS, :]
    o_ref[:, FULL_ROWS:, :] = (
        tail_ref[:, : SEQ - FULL_ROWS, :] + pos[None, FULL_ROWS:, :]
    )


def _pos_add(out_sc, tail_tok, position_table):
    return pl.pallas_call(
        _add_body,
        out_shape=jax.ShapeDtypeStruct((BATCH, SEQ, HIDDEN), jnp.float32),
        grid=(BATCH // BATCH_BLOCK,),
        in_specs=[
            pl.BlockSpec((BATCH_BLOCK, SEQ, HIDDEN), lambda b: (b, 0, 0)),
            pl.BlockSpec((BATCH_BLOCK, TAIL, HIDDEN), lambda b: (b, 0, 0)),
            pl.BlockSpec((SEQ, HIDDEN), lambda b: (0, 0)),
        ],
        out_specs=pl.BlockSpec((BATCH_BLOCK, SEQ, HIDDEN), lambda b: (b, 0, 0)),
        input_output_aliases={0: 0},
    )(out_sc, tail_tok, position_table)


@jax.jit
def _embed(ids, token_table, position_table):
    tail_ids = jnp.pad(ids[:, FULL_ROWS:], ((0, 0), (0, TAIL - (SEQ - FULL_ROWS))))
    mesh = plsc.VectorSubcoreMesh(
        core_axis_name="c", subcore_axis_name="s",
        num_cores=NUM_CORES, num_subcores=NUM_SUBCORES,
    )
    f = pl.kernel(
        _embed_body,
        out_type=(
            jax.ShapeDtypeStruct((BATCH, SEQ, HIDDEN), jnp.float32),
            jax.ShapeDtypeStruct((BATCH, TAIL, HIDDEN), jnp.float32),
        ),
        mesh=mesh,
        scratch_types=[
            pltpu.VMEM((SEQ,), jnp.int32),
            pltpu.VMEM((TAIL,), jnp.int32),
            pltpu.VMEM((SEQ, HIDDEN), jnp.float32),
            pltpu.VMEM((TAIL, HIDDEN), jnp.float32),
            pltpu.SemaphoreType.DMA,
            pltpu.SemaphoreType.DMA,
        ],
    )
    out_sc, tail_tok = f(ids, tail_ids, token_table)
    return _pos_add(out_sc, tail_tok, position_table)


def kernel(input_ids, token_table, position_table):
    ids = input_ids.astype(jnp.int32)
    return _embed(ids, token_table, position_table)


# speedup vs baseline: 1.1292x; 1.1292x over previous
import jax
import jax.numpy as jnp
from jax import lax
from jax.experimental import pallas as pl
from jax.experimental.pallas import tpu as pltpu
from jax.experimental.pallas import tpu_sc as plsc

VOCAB = 49408
HIDDEN = 768
SEQ = 77
BATCH = 1024
LANES = 16
NVEC = HIDDEN // LANES

NUM_CORES = 2
NUM_SUBCORES = 16
NUM_WORKERS = NUM_CORES * NUM_SUBCORES
BATCHES_PER_WORKER = BATCH // NUM_WORKERS

FULL_ROWS = 72
CHUNK_A = 40
CHUNK_B = 32
TAIL = 8
BATCH_BLOCK = 8


def _embed_body(ids_hbm, tids_hbm, tok_hbm, pos_hbm, out_hbm, tail_hbm,
                pos_v, idx_v, tidx_v, buf_a, buf_b, buf_t,
                gsem_a, gsem_b, gsem_t, ssem_a, ssem_b, ssem_t):
    cid = lax.axis_index("c")
    sid = lax.axis_index("s")
    wid = sid * NUM_CORES + cid
    base_b = wid * BATCHES_PER_WORKER

    pltpu.sync_copy(pos_hbm.at[pl.ds(0, FULL_ROWS)], pos_v)

    def add_pos(buf, nrows, pos_off):
        def row_body(r, carry):
            for c in range(NVEC):
                sl = pl.ds(c * LANES, LANES)
                x = pos_v[pos_off + r, sl]
                plsc.addupdate(buf.at[r, sl], x)
            return carry
        lax.fori_loop(0, nrows, row_body, 0)

    def drain_stores(gb):
        pltpu.make_async_copy(buf_a, out_hbm.at[gb, pl.ds(0, CHUNK_A)],
                              ssem_a).wait()
        pltpu.make_async_copy(buf_b, out_hbm.at[gb, pl.ds(CHUNK_A, CHUNK_B)],
                              ssem_b).wait()
        pltpu.make_async_copy(buf_t, tail_hbm.at[gb], ssem_t).wait()

    def batch_body(i, carry):
        gb = base_b + i
        pltpu.sync_copy(ids_hbm.at[gb], idx_v)
        pltpu.sync_copy(tids_hbm.at[gb], tidx_v)

        @pl.when(i > 0)
        def _():
            drain_stores(gb)

        ga = pltpu.async_copy(tok_hbm.at[idx_v.at[pl.ds(0, CHUNK_A)]],
                              buf_a, gsem_a)
        gb_ = pltpu.async_copy(tok_hbm.at[idx_v.at[pl.ds(CHUNK_A, CHUNK_B)]],
                               buf_b, gsem_b)
        gt = pltpu.async_copy(tok_hbm.at[tidx_v], buf_t, gsem_t)

        ga.wait()
        add_pos(buf_a, CHUNK_A, 0)
        pltpu.async_copy(buf_a, out_hbm.at[gb, pl.ds(0, CHUNK_A)], ssem_a)
        gb_.wait()
        add_pos(buf_b, CHUNK_B, CHUNK_A)
        pltpu.async_copy(buf_b, out_hbm.at[gb, pl.ds(CHUNK_A, CHUNK_B)], ssem_b)
        gt.wait()
        pltpu.async_copy(buf_t, tail_hbm.at[gb], ssem_t)
        return carry

    lax.fori_loop(0, BATCHES_PER_WORKER, batch_body, 0)
    drain_stores(base_b + BATCHES_PER_WORKER - 1)


def _tail_body(x_ref, tail_ref, pos_ref, o_ref):
    o_ref[...] = tail_ref[...] + pos_ref[...][None, :, :]


def _tail_fix(out_sc, tail_tok, position_table):
    return pl.pallas_call(
        _tail_body,
        out_shape=jax.ShapeDtypeStruct((BATCH, SEQ, HIDDEN), jnp.float32),
        grid=(BATCH // BATCH_BLOCK,),
        in_specs=[
            pl.BlockSpec((1, TAIL, HIDDEN), lambda b: (b, 9, 0)),
            pl.BlockSpec((BATCH_BLOCK, TAIL, HIDDEN), lambda b: (b, 0, 0)),
            pl.BlockSpec((TAIL, HIDDEN), lambda b: (9, 0)),
        ],
        out_specs=pl.BlockSpec((BATCH_BLOCK, TAIL, HIDDEN), lambda b: (b, 9, 0)),
        input_output_aliases={0: 0},
    )(out_sc, tail_tok, position_table)


@jax.jit
def _embed(ids, token_table, position_table):
    tail_ids = jnp.pad(ids[:, FULL_ROWS:], ((0, 0), (0, TAIL - (SEQ - FULL_ROWS))))
    mesh = plsc.VectorSubcoreMesh(
        core_axis_name="c", subcore_axis_name="s",
        num_cores=NUM_CORES, num_subcores=NUM_SUBCORES,
    )
    f = pl.kernel(
        _embed_body,
        out_type=(
            jax.ShapeDtypeStruct((BATCH, SEQ, HIDDEN), jnp.float32),
            jax.ShapeDtypeStruct((BATCH, TAIL, HIDDEN), jnp.float32),
        ),
        mesh=mesh,
        scratch_types=[
            pltpu.VMEM((FULL_ROWS, HIDDEN), jnp.float32),
            pltpu.VMEM((SEQ,), jnp.int32),
            pltpu.VMEM((TAIL,), jnp.int32),
            pltpu.VMEM((CHUNK_A, HIDDEN), jnp.float32),
            pltpu.VMEM((CHUNK_B, HIDDEN), jnp.float32),
            pltpu.VMEM((TAIL, HIDDEN), jnp.float32),
            pltpu.SemaphoreType.DMA,
            pltpu.SemaphoreType.DMA,
            pltpu.SemaphoreType.DMA,
            pltpu.SemaphoreType.DMA,
            pltpu.SemaphoreType.DMA,
            pltpu.SemaphoreType.DMA,
        ],
    )
    out_sc, tail_tok = f(ids, tail_ids, token_table, position_table)
    return _tail_fix(out_sc, tail_tok, position_table)


def kernel(input_ids, token_table, position_table):
    ids = input_ids.astype(jnp.int32)
    return _embed(ids, token_table, position_table)
